# Initial kernel scaffold; baseline (speedup 1.0000x reference)
import jax
import jax.numpy as jnp
from jax.experimental import pallas as pl


def kernel(x, emb):
    raise NotImplementedError("write your pallas kernel here")


# speedup vs baseline: 2.3296x; 2.3296x over previous
import jax
import jax.numpy as jnp
from jax.experimental import pallas as pl

SEQ_LEN = 8192
D_MODEL = 1024
BLOCK_ROWS = 512


def _add_body(x_ref, emb_ref, out_ref):
    out_ref[...] = x_ref[...] + emb_ref[...]


def kernel(x, emb):
    grid = (SEQ_LEN // BLOCK_ROWS,)
    spec = pl.BlockSpec((BLOCK_ROWS, D_MODEL), lambda i: (i, 0))
    return pl.pallas_call(
        _add_body,
        grid=grid,
        in_specs=[spec, spec],
        out_specs=spec,
        out_shape=jax.ShapeDtypeStruct((SEQ_LEN, D_MODEL), jnp.float32),
    )(x, emb[:SEQ_LEN])
